# baseline (speedup 1.0000x reference)
import functools

import jax
import jax.numpy as jnp
from jax import lax
from jax.experimental import pallas as pl
from jax.experimental.pallas import tpu as pltpu
from jax.experimental.pallas import tpu_sc as plsc

N = 10000
D = 128
E = 320000

NC = 2
NS = 16
NW = NC * NS

K = 64
EW = E // NW
CH = EW // K
KT = EW - CH * K
NB = 4
GA = 2
SB = 2
NQ = 12
PF = NQ - SB
assert GA + SB <= NB and NQ % NB == 0 and CH % NQ == 0 and PF >= GA

N_PAD = 10112
RW = N_PAD // NS

_sc_mesh = plsc.VectorSubcoreMesh(
    core_axis_name="c", subcore_axis_name="s", num_cores=NC, num_subcores=NS
)


def _make_sc_agg(with_deg):
    def body(*refs):
        if with_deg:
            (x_hbm, src_hbm, dst_hbm, zrows_hbm, zdeg_hbm, acc_out, deg_out,
             acc_sh, deg_sh, *rest) = refs
        else:
            (x_hbm, src_hbm, dst_hbm, zrows_hbm, acc_out,
             acc_sh, *rest) = refs
        rows = rest[0:NB]
        sidx = rest[NB:NB + NQ]
        didx = rest[NB + NQ:NB + 2 * NQ]
        stail, dtail, rtail = rest[NB + 2 * NQ:NB + 2 * NQ + 3]
        p = NB + 2 * NQ + 3
        if with_deg:
            ones_v, deg_v = rest[p:p + 2]
            p += 2
        sems = rest[p:]
        gsem = sems[0:NB]
        ssem = sems[NB:2 * NB]
        isem = sems[2 * NB:2 * NB + NQ]
        dsem = sems[2 * NB + NQ:] if with_deg else None

        cid = lax.axis_index("c")
        sid = lax.axis_index("s")
        wid = cid * NS + sid

        pltpu.sync_copy(zrows_hbm.at[pl.ds(sid * RW, RW)],
                        acc_sh.at[pl.ds(sid * RW, RW)])
        if with_deg:
            pltpu.sync_copy(zdeg_hbm.at[pl.ds(sid * RW, RW)], deg_v)
            pltpu.sync_copy(deg_v, deg_sh.at[pl.ds(sid * RW, RW)])
            for j in range(128 // 16):
                ones_v[pl.ds(j * 16, 16)] = jnp.ones((16,), jnp.float32)

        def i_start(c, q):
            off = wid * EW + c * K
            pltpu.async_copy(src_hbm.at[pl.ds(off, K)], sidx[q], isem[q])
            pltpu.async_copy(dst_hbm.at[pl.ds(off, K)], didx[q], isem[q])

        def i_wait(c, q):
            off = wid * EW + c * K
            pltpu.make_async_copy(src_hbm.at[pl.ds(off, K)], sidx[q],
                                  isem[q]).wait()
            pltpu.make_async_copy(dst_hbm.at[pl.ds(off, K)], didx[q],
                                  isem[q]).wait()

        def g_start(c, b, q):
            pltpu.async_copy(x_hbm.at[sidx[q]], rows[b], gsem[b])

        def g_wait(c, b, q):
            pltpu.make_async_copy(x_hbm.at[sidx[q]], rows[b], gsem[b]).wait()

        def s_start(c, b, q):
            pltpu.async_copy(rows[b], acc_sh.at[didx[q]], ssem[b], add=True)

        def s_wait(c, b, q):
            pltpu.make_async_copy(rows[b], acc_sh.at[didx[q]], ssem[b]).wait()

        def d_start(c, b, q):
            pltpu.async_copy(ones_v.at[pl.ds(0, K)], deg_sh.at[didx[q]],
                             dsem[b], add=True)

        def d_wait(c, b, q):
            pltpu.make_async_copy(ones_v.at[pl.ds(0, K)], deg_sh.at[didx[q]],
                                  dsem[b]).wait()

        for q in range(PF):
            i_start(q, q)
        plsc.subcore_barrier()
        for g in range(GA):
            i_wait(g, g)
            g_start(g, g % NB, g)

        def step(t, carry):
            for j in range(NQ):
                c = t * NQ + j

                g_wait(c, j % NB, j)
                s_start(c, j % NB, j)
                if with_deg:
                    d_start(c, j % NB, j)

                def retire():
                    s_wait(c - SB, (j - SB) % NB, (j - SB) % NQ)
                    if with_deg:
                        d_wait(c - SB, (j - SB) % NB, (j - SB) % NQ)
                if j >= SB:
                    retire()
                else:
                    pl.when(t > 0)(retire)

                @pl.when(c + PF < CH)
                def _():
                    i_start(c + PF, (j + PF) % NQ)

                @pl.when(c + GA < CH)
                def _():
                    i_wait(c + GA, (j + GA) % NQ)
                    g_start(c + GA, (j + GA) % NB, (j + GA) % NQ)
            return carry

        lax.fori_loop(0, CH // NQ, step, 0)

        toff = wid * EW + CH * K
        pltpu.sync_copy(src_hbm.at[pl.ds(toff, KT)], stail)
        pltpu.sync_copy(dst_hbm.at[pl.ds(toff, KT)], dtail)
        pltpu.async_copy(x_hbm.at[stail], rtail, gsem[0]).wait()
        pltpu.sync_copy(rtail, acc_sh.at[dtail], add=True)
        if with_deg:
            pltpu.sync_copy(ones_v.at[pl.ds(0, KT)], deg_sh.at[dtail],
                            add=True)

        for c in range(CH - SB, CH):
            s_wait(c, c % NB, c % NQ)
            if with_deg:
                d_wait(c, c % NB, c % NQ)

        plsc.subcore_barrier()

        pltpu.sync_copy(acc_sh.at[pl.ds(sid * RW, RW)],
                        acc_out.at[cid, pl.ds(sid * RW, RW)])
        if with_deg:
            pltpu.sync_copy(deg_sh.at[pl.ds(sid * RW, RW)], deg_v)
            pltpu.sync_copy(deg_v,
                            deg_out.at[pl.ds(cid * N_PAD + sid * RW, RW)])

    out_type = [jax.ShapeDtypeStruct((NC, N_PAD, D), jnp.float32)]
    scratch = [pltpu.VMEM_SHARED((N_PAD, D), jnp.float32)]
    if with_deg:
        out_type.append(jax.ShapeDtypeStruct((NC * N_PAD,), jnp.float32))
        scratch.append(pltpu.VMEM_SHARED((N_PAD,), jnp.float32))
    scratch += [pltpu.VMEM((K, D), jnp.float32) for _ in range(NB)]
    scratch += [pltpu.VMEM((K,), jnp.int32) for _ in range(2 * NQ)]
    scratch += [
        pltpu.VMEM((KT,), jnp.int32),
        pltpu.VMEM((KT,), jnp.int32),
        pltpu.VMEM((KT, D), jnp.float32),
    ]
    if with_deg:
        scratch += [
            pltpu.VMEM((128,), jnp.float32),
            pltpu.VMEM((RW,), jnp.float32),
        ]
    nsem = 2 * NB + NQ + (NB if with_deg else 0)
    scratch += [pltpu.SemaphoreType.DMA for _ in range(nsem)]

    return pl.kernel(
        body,
        out_type=out_type,
        mesh=_sc_mesh,
        scratch_types=scratch,
    )


_sc_agg_deg = _make_sc_agg(True)
_sc_agg = _make_sc_agg(False)


_TC_R = 2000


def _tc_self_body(h_ref, ws_ref, b_ref, out_ref):
    out_ref[...] = (jnp.dot(h_ref[...], ws_ref[...],
                            preferred_element_type=jnp.float32,
                            precision=lax.Precision.HIGHEST)
                    + b_ref[...])


def _tc_self(h, w_self, b):
    return pl.pallas_call(
        _tc_self_body,
        grid=(N // _TC_R,),
        in_specs=[
            pl.BlockSpec((_TC_R, D), lambda i: (i, 0)),
            pl.BlockSpec((D, D), lambda i: (0, 0)),
            pl.BlockSpec((1, D), lambda i: (0, 0)),
        ],
        out_specs=pl.BlockSpec((_TC_R, D), lambda i: (i, 0)),
        out_shape=jax.ShapeDtypeStruct((N, D), jnp.float32),
    )(h, w_self, b)


def _tc_combine_body(acc_ref, deg_ref, self_ref, wn_ref, out_ref):
    p = acc_ref[0] + acc_ref[1]
    d = jnp.maximum(deg_ref[0] + deg_ref[1], 1.0)
    agg = p / d
    y = jnp.dot(agg, wn_ref[...], preferred_element_type=jnp.float32,
                precision=lax.Precision.HIGHEST) + self_ref[...]
    out_ref[...] = jnp.maximum(y, 0.0)


def _tc_combine(acc, deg, selfterm, w_neigh):
    return pl.pallas_call(
        _tc_combine_body,
        grid=(N // _TC_R,),
        in_specs=[
            pl.BlockSpec((NC, _TC_R, D), lambda i: (0, i, 0)),
            pl.BlockSpec((NC, _TC_R, 1), lambda i: (0, i, 0)),
            pl.BlockSpec((_TC_R, D), lambda i: (i, 0)),
            pl.BlockSpec((D, D), lambda i: (0, 0)),
        ],
        out_specs=pl.BlockSpec((_TC_R, D), lambda i: (i, 0)),
        out_shape=jax.ShapeDtypeStruct((N, D), jnp.float32),
    )(acc, deg, selfterm, w_neigh)


def kernel(x, edge_index, W_self1, W_neigh1, b1, W_self2, W_neigh2, b2):
    e = edge_index.astype(jnp.int32)
    src = e[0]
    dst = e[1]
    zrows = jnp.zeros((N_PAD, D), jnp.float32)
    zdeg = jnp.zeros((N_PAD,), jnp.float32)
    b1r = b1.reshape(1, D)
    b2r = b2.reshape(1, D)

    acc1, deg = _sc_agg_deg(x, src, dst, zrows, zdeg)
    self1 = _tc_self(x, W_self1, b1r)
    deg3 = deg.reshape(NC, N_PAD, 1)
    h1 = _tc_combine(acc1, deg3, self1, W_neigh1)
    (acc2,) = _sc_agg(h1, src, dst, zrows)
    self2 = _tc_self(h1, W_self2, b2r)
    h2 = _tc_combine(acc2, deg3, self2, W_neigh2)
    return h2


# speedup vs baseline: 1.1572x; 1.1572x over previous
import functools

import jax
import jax.numpy as jnp
from jax import lax
from jax.experimental import pallas as pl
from jax.experimental.pallas import tpu as pltpu
from jax.experimental.pallas import tpu_sc as plsc

N = 10000
D = 128
E = 320000

NC = 2
NS = 16
NW = NC * NS

K = 104
EW = E // NW
CH = EW // K
KT = EW - CH * K
NB = 3
GA = 2
SB = 1
NQ = 6
PF = NQ - SB
assert GA + SB <= NB and NQ % NB == 0 and CH % NQ == 0 and PF >= GA

N_PAD = 10112
RW = N_PAD // NS

_sc_mesh = plsc.VectorSubcoreMesh(
    core_axis_name="c", subcore_axis_name="s", num_cores=NC, num_subcores=NS
)


def _make_sc_agg(with_deg):
    def body(*refs):
        if with_deg:
            (x_hbm, src_hbm, dst_hbm, zrows_hbm, zdeg_hbm, acc_out, deg_out,
             acc_sh, deg_sh, *rest) = refs
        else:
            (x_hbm, src_hbm, dst_hbm, zrows_hbm, acc_out,
             acc_sh, *rest) = refs
        rows = rest[0:NB]
        sidx = rest[NB:NB + NQ]
        didx = rest[NB + NQ:NB + 2 * NQ]
        stail, dtail, rtail = rest[NB + 2 * NQ:NB + 2 * NQ + 3]
        p = NB + 2 * NQ + 3
        if with_deg:
            ones_v, deg_v = rest[p:p + 2]
            p += 2
        sems = rest[p:]
        gsem = sems[0:NB]
        ssem = sems[NB:2 * NB]
        isem = sems[2 * NB:2 * NB + NQ]
        dsem = sems[2 * NB + NQ:] if with_deg else None

        cid = lax.axis_index("c")
        sid = lax.axis_index("s")
        wid = cid * NS + sid

        pltpu.sync_copy(zrows_hbm.at[pl.ds(sid * RW, RW)],
                        acc_sh.at[pl.ds(sid * RW, RW)])
        if with_deg:
            pltpu.sync_copy(zdeg_hbm.at[pl.ds(sid * RW, RW)], deg_v)
            pltpu.sync_copy(deg_v, deg_sh.at[pl.ds(sid * RW, RW)])
            for j in range(128 // 16):
                ones_v[pl.ds(j * 16, 16)] = jnp.ones((16,), jnp.float32)

        def i_start(c, q):
            off = wid * EW + c * K
            pltpu.async_copy(src_hbm.at[pl.ds(off, K)], sidx[q], isem[q])
            pltpu.async_copy(dst_hbm.at[pl.ds(off, K)], didx[q], isem[q])

        def i_wait(c, q):
            off = wid * EW + c * K
            pltpu.make_async_copy(src_hbm.at[pl.ds(off, K)], sidx[q],
                                  isem[q]).wait()
            pltpu.make_async_copy(dst_hbm.at[pl.ds(off, K)], didx[q],
                                  isem[q]).wait()

        def g_start(c, b, q):
            pltpu.async_copy(x_hbm.at[sidx[q]], rows[b], gsem[b])

        def g_wait(c, b, q):
            pltpu.make_async_copy(x_hbm.at[sidx[q]], rows[b], gsem[b]).wait()

        def s_start(c, b, q):
            pltpu.async_copy(rows[b], acc_sh.at[didx[q]], ssem[b], add=True)

        def s_wait(c, b, q):
            pltpu.make_async_copy(rows[b], acc_sh.at[didx[q]], ssem[b]).wait()

        def d_start(c, b, q):
            pltpu.async_copy(ones_v.at[pl.ds(0, K)], deg_sh.at[didx[q]],
                             dsem[b], add=True)

        def d_wait(c, b, q):
            pltpu.make_async_copy(ones_v.at[pl.ds(0, K)], deg_sh.at[didx[q]],
                                  dsem[b]).wait()

        for q in range(PF):
            i_start(q, q)
        plsc.subcore_barrier()
        for g in range(GA):
            i_wait(g, g)
            g_start(g, g % NB, g)

        def step(t, carry):
            for j in range(NQ):
                c = t * NQ + j

                g_wait(c, j % NB, j)
                s_start(c, j % NB, j)
                if with_deg:
                    d_start(c, j % NB, j)

                def retire():
                    s_wait(c - SB, (j - SB) % NB, (j - SB) % NQ)
                    if with_deg:
                        d_wait(c - SB, (j - SB) % NB, (j - SB) % NQ)
                if j >= SB:
                    retire()
                else:
                    pl.when(t > 0)(retire)

                @pl.when(c + PF < CH)
                def _():
                    i_start(c + PF, (j + PF) % NQ)

                @pl.when(c + GA < CH)
                def _():
                    i_wait(c + GA, (j + GA) % NQ)
                    g_start(c + GA, (j + GA) % NB, (j + GA) % NQ)
            return carry

        lax.fori_loop(0, CH // NQ, step, 0)

        toff = wid * EW + CH * K
        pltpu.sync_copy(src_hbm.at[pl.ds(toff, KT)], stail)
        pltpu.sync_copy(dst_hbm.at[pl.ds(toff, KT)], dtail)
        pltpu.async_copy(x_hbm.at[stail], rtail, gsem[0]).wait()
        pltpu.sync_copy(rtail, acc_sh.at[dtail], add=True)
        if with_deg:
            pltpu.sync_copy(ones_v.at[pl.ds(0, KT)], deg_sh.at[dtail],
                            add=True)

        for c in range(CH - SB, CH):
            s_wait(c, c % NB, c % NQ)
            if with_deg:
                d_wait(c, c % NB, c % NQ)

        plsc.subcore_barrier()

        pltpu.sync_copy(acc_sh.at[pl.ds(sid * RW, RW)],
                        acc_out.at[cid, pl.ds(sid * RW, RW)])
        if with_deg:
            pltpu.sync_copy(deg_sh.at[pl.ds(sid * RW, RW)], deg_v)
            pltpu.sync_copy(deg_v,
                            deg_out.at[pl.ds(cid * N_PAD + sid * RW, RW)])

    out_type = [jax.ShapeDtypeStruct((NC, N_PAD, D), jnp.float32)]
    scratch = [pltpu.VMEM_SHARED((N_PAD, D), jnp.float32)]
    if with_deg:
        out_type.append(jax.ShapeDtypeStruct((NC * N_PAD,), jnp.float32))
        scratch.append(pltpu.VMEM_SHARED((N_PAD,), jnp.float32))
    scratch += [pltpu.VMEM((K, D), jnp.float32) for _ in range(NB)]
    scratch += [pltpu.VMEM((K,), jnp.int32) for _ in range(2 * NQ)]
    scratch += [
        pltpu.VMEM((KT,), jnp.int32),
        pltpu.VMEM((KT,), jnp.int32),
        pltpu.VMEM((KT, D), jnp.float32),
    ]
    if with_deg:
        scratch += [
            pltpu.VMEM((128,), jnp.float32),
            pltpu.VMEM((RW,), jnp.float32),
        ]
    nsem = 2 * NB + NQ + (NB if with_deg else 0)
    scratch += [pltpu.SemaphoreType.DMA for _ in range(nsem)]

    return pl.kernel(
        body,
        out_type=out_type,
        mesh=_sc_mesh,
        scratch_types=scratch,
    )


_sc_agg_deg = _make_sc_agg(True)
_sc_agg = _make_sc_agg(False)


_TC_R = 2000


def _tc_self_body(h_ref, ws_ref, b_ref, out_ref):
    out_ref[...] = (jnp.dot(h_ref[...], ws_ref[...],
                            preferred_element_type=jnp.float32,
                            precision=lax.Precision.HIGHEST)
                    + b_ref[...])


def _tc_self(h, w_self, b):
    return pl.pallas_call(
        _tc_self_body,
        grid=(N // _TC_R,),
        in_specs=[
            pl.BlockSpec((_TC_R, D), lambda i: (i, 0)),
            pl.BlockSpec((D, D), lambda i: (0, 0)),
            pl.BlockSpec((1, D), lambda i: (0, 0)),
        ],
        out_specs=pl.BlockSpec((_TC_R, D), lambda i: (i, 0)),
        out_shape=jax.ShapeDtypeStruct((N, D), jnp.float32),
    )(h, w_self, b)


def _tc_combine_body(acc_ref, deg_ref, self_ref, wn_ref, out_ref):
    p = acc_ref[0] + acc_ref[1]
    d = jnp.maximum(deg_ref[0] + deg_ref[1], 1.0)
    agg = p / d
    y = jnp.dot(agg, wn_ref[...], preferred_element_type=jnp.float32,
                precision=lax.Precision.HIGHEST) + self_ref[...]
    out_ref[...] = jnp.maximum(y, 0.0)


def _tc_combine(acc, deg, selfterm, w_neigh):
    return pl.pallas_call(
        _tc_combine_body,
        grid=(N // _TC_R,),
        in_specs=[
            pl.BlockSpec((NC, _TC_R, D), lambda i: (0, i, 0)),
            pl.BlockSpec((NC, _TC_R, 1), lambda i: (0, i, 0)),
            pl.BlockSpec((_TC_R, D), lambda i: (i, 0)),
            pl.BlockSpec((D, D), lambda i: (0, 0)),
        ],
        out_specs=pl.BlockSpec((_TC_R, D), lambda i: (i, 0)),
        out_shape=jax.ShapeDtypeStruct((N, D), jnp.float32),
    )(acc, deg, selfterm, w_neigh)


def kernel(x, edge_index, W_self1, W_neigh1, b1, W_self2, W_neigh2, b2):
    e = edge_index.astype(jnp.int32)
    src = e[0]
    dst = e[1]
    zrows = jnp.zeros((N_PAD, D), jnp.float32)
    zdeg = jnp.zeros((N_PAD,), jnp.float32)
    b1r = b1.reshape(1, D)
    b2r = b2.reshape(1, D)

    acc1, deg = _sc_agg_deg(x, src, dst, zrows, zdeg)
    self1 = _tc_self(x, W_self1, b1r)
    deg3 = deg.reshape(NC, N_PAD, 1)
    h1 = _tc_combine(acc1, deg3, self1, W_neigh1)
    (acc2,) = _sc_agg(h1, src, dst, zrows)
    self2 = _tc_self(h1, W_self2, b2r)
    h2 = _tc_combine(acc2, deg3, self2, W_neigh2)
    return h2


# speedup vs baseline: 1.1627x; 1.0047x over previous
import functools

import jax
import jax.numpy as jnp
from jax import lax
from jax.experimental import pallas as pl
from jax.experimental.pallas import tpu as pltpu
from jax.experimental.pallas import tpu_sc as plsc

N = 10000
D = 128
E = 320000

NC = 2
NS = 16
NW = NC * NS

K = 104
EW = E // NW
CH = EW // K
KT = EW - CH * K
NB = 3
GA = 2
SB = 1
NQ = 6
PF = NQ - SB
assert GA + SB <= NB and NQ % NB == 0 and CH % NQ == 0 and PF >= GA

N_PAD = 10112
RW = N_PAD // NS

_sc_mesh = plsc.VectorSubcoreMesh(
    core_axis_name="c", subcore_axis_name="s", num_cores=NC, num_subcores=NS
)


def _make_sc_agg(with_deg):
    def body(*refs):
        if with_deg:
            (x_hbm, src_hbm, dst_hbm, zrows_hbm, zdeg_hbm, acc_out, deg_out,
             acc_sh, deg_sh, *rest) = refs
        else:
            (x_hbm, src_hbm, dst_hbm, zrows_hbm, acc_out,
             acc_sh, *rest) = refs
        rows = rest[0:NB]
        sidx = rest[NB:NB + NQ]
        didx = rest[NB + NQ:NB + 2 * NQ]
        stail, dtail, rtail = rest[NB + 2 * NQ:NB + 2 * NQ + 3]
        p = NB + 2 * NQ + 3
        if with_deg:
            ones_v, deg_v = rest[p:p + 2]
            p += 2
        sems = rest[p:]
        gsem = sems[0:NB]
        ssem = sems[NB:2 * NB]
        isem = sems[2 * NB:2 * NB + NQ]
        dsem = sems[2 * NB + NQ:] if with_deg else None

        cid = lax.axis_index("c")
        sid = lax.axis_index("s")
        wid = cid * NS + sid

        pltpu.sync_copy(zrows_hbm.at[pl.ds(sid * RW, RW)],
                        acc_sh.at[pl.ds(sid * RW, RW)])
        if with_deg:
            pltpu.sync_copy(zdeg_hbm.at[pl.ds(sid * RW, RW)], deg_v)
            pltpu.sync_copy(deg_v, deg_sh.at[pl.ds(sid * RW, RW)])
            for j in range(128 // 16):
                ones_v[pl.ds(j * 16, 16)] = jnp.ones((16,), jnp.float32)

        def i_start(c, q):
            off = wid * EW + c * K
            pltpu.async_copy(src_hbm.at[pl.ds(off, K)], sidx[q], isem[q])
            pltpu.async_copy(dst_hbm.at[pl.ds(off, K)], didx[q], isem[q])

        def i_wait(c, q):
            off = wid * EW + c * K
            pltpu.make_async_copy(src_hbm.at[pl.ds(off, K)], sidx[q],
                                  isem[q]).wait()
            pltpu.make_async_copy(dst_hbm.at[pl.ds(off, K)], didx[q],
                                  isem[q]).wait()

        def g_start(c, b, q):
            pltpu.async_copy(x_hbm.at[sidx[q]], rows[b], gsem[b])

        def g_wait(c, b, q):
            pltpu.make_async_copy(x_hbm.at[sidx[q]], rows[b], gsem[b]).wait()

        def s_start(c, b, q):
            pltpu.async_copy(rows[b], acc_sh.at[didx[q]], ssem[b], add=True)

        def s_wait(c, b, q):
            pltpu.make_async_copy(rows[b], acc_sh.at[didx[q]], ssem[b]).wait()

        def d_start(c, b, q):
            pltpu.async_copy(ones_v.at[pl.ds(0, K)], deg_sh.at[didx[q]],
                             dsem[b], add=True)

        def d_wait(c, b, q):
            pltpu.make_async_copy(ones_v.at[pl.ds(0, K)], deg_sh.at[didx[q]],
                                  dsem[b]).wait()

        for q in range(PF):
            i_start(q, q)
        plsc.subcore_barrier()
        for g in range(GA):
            i_wait(g, g)
            g_start(g, g % NB, g)

        def step(t, carry):
            for j in range(NQ):
                c = t * NQ + j

                g_wait(c, j % NB, j)
                s_start(c, j % NB, j)
                if with_deg:
                    d_start(c, j % NB, j)

                def retire():
                    s_wait(c - SB, (j - SB) % NB, (j - SB) % NQ)
                    if with_deg:
                        d_wait(c - SB, (j - SB) % NB, (j - SB) % NQ)
                if j >= SB:
                    retire()
                else:
                    pl.when(t > 0)(retire)

                @pl.when(c + PF < CH)
                def _():
                    i_start(c + PF, (j + PF) % NQ)

                @pl.when(c + GA < CH)
                def _():
                    i_wait(c + GA, (j + GA) % NQ)
                    g_start(c + GA, (j + GA) % NB, (j + GA) % NQ)
            return carry

        lax.fori_loop(0, CH // NQ, step, 0)

        toff = wid * EW + CH * K
        pltpu.sync_copy(src_hbm.at[pl.ds(toff, KT)], stail)
        pltpu.sync_copy(dst_hbm.at[pl.ds(toff, KT)], dtail)
        pltpu.async_copy(x_hbm.at[stail], rtail, gsem[0]).wait()
        pltpu.sync_copy(rtail, acc_sh.at[dtail], add=True)
        if with_deg:
            pltpu.sync_copy(ones_v.at[pl.ds(0, KT)], deg_sh.at[dtail],
                            add=True)

        for c in range(CH - SB, CH):
            s_wait(c, c % NB, c % NQ)
            if with_deg:
                d_wait(c, c % NB, c % NQ)

        plsc.subcore_barrier()

        pltpu.sync_copy(acc_sh.at[pl.ds(sid * RW, RW)],
                        acc_out.at[cid, pl.ds(sid * RW, RW)])
        if with_deg:
            pltpu.sync_copy(deg_sh.at[pl.ds(sid * RW, RW)], deg_v)
            pltpu.sync_copy(deg_v,
                            deg_out.at[pl.ds(cid * N_PAD + sid * RW, RW)])

    out_type = [jax.ShapeDtypeStruct((NC, N_PAD, D), jnp.float32)]
    scratch = [pltpu.VMEM_SHARED((N_PAD, D), jnp.float32)]
    if with_deg:
        out_type.append(jax.ShapeDtypeStruct((NC * N_PAD,), jnp.float32))
        scratch.append(pltpu.VMEM_SHARED((N_PAD,), jnp.float32))
    scratch += [pltpu.VMEM((K, D), jnp.float32) for _ in range(NB)]
    scratch += [pltpu.VMEM((K,), jnp.int32) for _ in range(2 * NQ)]
    scratch += [
        pltpu.VMEM((KT,), jnp.int32),
        pltpu.VMEM((KT,), jnp.int32),
        pltpu.VMEM((KT, D), jnp.float32),
    ]
    if with_deg:
        scratch += [
            pltpu.VMEM((128,), jnp.float32),
            pltpu.VMEM((RW,), jnp.float32),
        ]
    nsem = 2 * NB + NQ + (NB if with_deg else 0)
    scratch += [pltpu.SemaphoreType.DMA for _ in range(nsem)]

    return pl.kernel(
        body,
        out_type=out_type,
        mesh=_sc_mesh,
        scratch_types=scratch,
    )


_sc_agg_deg = _make_sc_agg(True)
_sc_agg = _make_sc_agg(False)


_TC_R = 2000


def _tc_self_body(h_ref, ws_ref, b_ref, out_ref):
    out_ref[...] = (jnp.dot(h_ref[...], ws_ref[...],
                            preferred_element_type=jnp.float32)
                    + b_ref[...])


def _tc_self(h, w_self, b):
    return pl.pallas_call(
        _tc_self_body,
        grid=(N // _TC_R,),
        in_specs=[
            pl.BlockSpec((_TC_R, D), lambda i: (i, 0)),
            pl.BlockSpec((D, D), lambda i: (0, 0)),
            pl.BlockSpec((1, D), lambda i: (0, 0)),
        ],
        out_specs=pl.BlockSpec((_TC_R, D), lambda i: (i, 0)),
        out_shape=jax.ShapeDtypeStruct((N, D), jnp.float32),
    )(h, w_self, b)


def _tc_combine_body(acc_ref, deg_ref, self_ref, wn_ref, out_ref):
    p = acc_ref[0] + acc_ref[1]
    d = jnp.maximum(deg_ref[0] + deg_ref[1], 1.0)
    agg = p / d
    y = jnp.dot(agg, wn_ref[...], preferred_element_type=jnp.float32) + self_ref[...]
    out_ref[...] = jnp.maximum(y, 0.0)


def _tc_combine(acc, deg, selfterm, w_neigh):
    return pl.pallas_call(
        _tc_combine_body,
        grid=(N // _TC_R,),
        in_specs=[
            pl.BlockSpec((NC, _TC_R, D), lambda i: (0, i, 0)),
            pl.BlockSpec((NC, _TC_R, 1), lambda i: (0, i, 0)),
            pl.BlockSpec((_TC_R, D), lambda i: (i, 0)),
            pl.BlockSpec((D, D), lambda i: (0, 0)),
        ],
        out_specs=pl.BlockSpec((_TC_R, D), lambda i: (i, 0)),
        out_shape=jax.ShapeDtypeStruct((N, D), jnp.float32),
    )(acc, deg, selfterm, w_neigh)


def kernel(x, edge_index, W_self1, W_neigh1, b1, W_self2, W_neigh2, b2):
    e = edge_index.astype(jnp.int32)
    src = e[0]
    dst = e[1]
    zrows = jnp.zeros((N_PAD, D), jnp.float32)
    zdeg = jnp.zeros((N_PAD,), jnp.float32)
    b1r = b1.reshape(1, D)
    b2r = b2.reshape(1, D)

    acc1, deg = _sc_agg_deg(x, src, dst, zrows, zdeg)
    self1 = _tc_self(x, W_self1, b1r)
    deg3 = deg.reshape(NC, N_PAD, 1)
    h1 = _tc_combine(acc1, deg3, self1, W_neigh1)
    (acc2,) = _sc_agg(h1, src, dst, zrows)
    self2 = _tc_self(h1, W_self2, b2r)
    h2 = _tc_combine(acc2, deg3, self2, W_neigh2)
    return h2
